# baseline (speedup 1.0000x reference)
import functools

import jax
import jax.numpy as jnp
from jax import lax
from jax.experimental import pallas as pl
from jax.experimental.pallas import tpu as pltpu
from jax.experimental.pallas import tpu_sc as plsc

N_ENT = 10000
N_EDGE = 160000
N_REL = 400
D = 128
NQ = 40000
ALPHA = 0.8
HALF = N_EDGE // 2

NC = 2
NS = 16
K = 128
NROWS = 10112
ROWS_PER_TILE = NROWS // NS
ZROWS = 158
DUMMY = N_ENT

HALFP = 81920
E_PER_TILE = HALFP // NS
ECH_TILE = E_PER_TILE // K
ECHUNKS = HALF // K
NQP = 40960
QCHUNKS = NQP // K

_mesh = plsc.VectorSubcoreMesh(core_axis_name="c", subcore_axis_name="s")


@functools.partial(
    pl.kernel,
    out_type=jax.ShapeDtypeStruct((NC, NROWS, D), jnp.float32),
    mesh=_mesh,
    scratch_types=[
        pltpu.VMEM_SHARED((NROWS, D), jnp.float32),
        pltpu.VMEM((8, K), jnp.int32),
        pltpu.VMEM((K, D), jnp.float32),
        pltpu.SemaphoreType.DMA,
    ],
)
def _cnt_kernel(dst2_hbm, z128_hbm, ones_hbm, c_hbm, cnt, didx, ones, sem):
    c = lax.axis_index("c")
    s = lax.axis_index("s")
    r0 = s * ROWS_PER_TILE
    for j in range(ROWS_PER_TILE // ZROWS):
        pltpu.sync_copy(z128_hbm, cnt.at[pl.ds(r0 + j * ZROWS, ZROWS)])
    pltpu.sync_copy(ones_hbm, ones)
    row0 = (c * NS + s) * ECH_TILE
    plsc.subcore_barrier()

    @pl.loop(0, ECH_TILE // 8)
    def _(g):
        pltpu.sync_copy(dst2_hbm.at[pl.ds(row0 + g * 8, 8)], didx)
        ds_ = [pltpu.async_copy(ones, cnt.at[didx.at[u]], sem, add=True)
               for u in range(8)]
        for d in ds_:
            d.wait()

    plsc.subcore_barrier()
    pltpu.sync_copy(cnt.at[pl.ds(r0, ROWS_PER_TILE)],
                    c_hbm.at[c, pl.ds(r0, ROWS_PER_TILE)])


@functools.partial(
    pl.kernel,
    out_type=jax.ShapeDtypeStruct((NC, NROWS, D), jnp.float32),
    mesh=_mesh,
    scratch_types=[
        pltpu.VMEM_SHARED((NROWS, D), jnp.float32),
        pltpu.VMEM_SHARED((N_REL, D), jnp.float32),
        pltpu.VMEM((K, D), jnp.float32),
        pltpu.VMEM((K, D), jnp.float32),
        pltpu.VMEM((3, K), jnp.int32),
        pltpu.VMEM((3, K), jnp.int32),
        pltpu.SemaphoreType.DMA,
        pltpu.SemaphoreType.DMA,
        pltpu.SemaphoreType.DMA,
    ],
)
def _edge_kernel(x_hbm, rneg_hbm, eidx_hbm, qidx_hbm, z128_hbm,
                 a_hbm, acc, rtab, xbuf, rbuf, ibuf0, ibuf1, semx, semr, semi):
    c = lax.axis_index("c")
    s = lax.axis_index("s")

    r0 = s * ROWS_PER_TILE
    for j in range(ROWS_PER_TILE // ZROWS):
        pltpu.sync_copy(z128_hbm, acc.at[pl.ds(r0 + j * ZROWS, ZROWS)])

    @pl.when(s == 0)
    def _():
        pltpu.sync_copy(rneg_hbm, rtab)

    plsc.subcore_barrier()

    pltpu.sync_copy(eidx_hbm.at[:, pl.ds(c * HALF + s * K, K)], ibuf0)

    @pl.loop(s, ECHUNKS, step=2 * NS)
    def _(t):
        for jj, pin, pout in ((t, ibuf0, ibuf1), (t + NS, ibuf1, ibuf0)):
            @pl.when(jj < ECHUNKS)
            def _():
                pf = pltpu.async_copy(
                    eidx_hbm.at[:, pl.ds(c * HALF + (jj + NS) * K, K)],
                    pout, semi)
                gx = pltpu.async_copy(x_hbm.at[pin.at[0]], xbuf, semx)
                gr = pltpu.async_copy(rtab.at[pin.at[1]], rbuf, semr)
                gx.wait()
                sx = pltpu.async_copy(xbuf, acc.at[pin.at[2]], semx, add=True)
                gr.wait()
                sr = pltpu.async_copy(rbuf, acc.at[pin.at[2]], semr, add=True)
                sx.wait()
                sr.wait()
                pf.wait()

    pltpu.sync_copy(qidx_hbm.at[c, :, pl.ds(s * K, K)], ibuf0)

    @pl.loop(s, QCHUNKS, step=2 * NS)
    def _(t):
        for jj, pin, pout in ((t, ibuf0, ibuf1), (t + NS, ibuf1, ibuf0)):
            @pl.when(jj < QCHUNKS)
            def _():
                pf = pltpu.async_copy(
                    qidx_hbm.at[c, :, pl.ds((jj + NS) * K, K)], pout, semi)
                gr = pltpu.async_copy(rtab.at[pin.at[0]], rbuf, semr)
                gx = pltpu.async_copy(x_hbm.at[pin.at[1]], xbuf, semx)
                gr.wait()
                gx.wait()

                @plsc.parallel_loop(0, K * D, step=16, unroll=8)
                def _(u):
                    i = u // D
                    k = u - i * D
                    xbuf[i, pl.ds(k, 16)] = (
                        xbuf[i, pl.ds(k, 16)] * rbuf[i, pl.ds(k, 16)]) * 0.25

                sc = pltpu.async_copy(xbuf, acc.at[pin.at[2]], semx, add=True)
                sc.wait()
                pf.wait()

    plsc.subcore_barrier()
    pltpu.sync_copy(acc.at[pl.ds(r0, ROWS_PER_TILE)],
                    a_hbm.at[c, pl.ds(r0, ROWS_PER_TILE)])


NGX = 7168
GCHUNKS = NGX // K


@functools.partial(
    pl.kernel,
    out_type=(jax.ShapeDtypeStruct((NGX, D), jnp.float32),
              jax.ShapeDtypeStruct((NGX, D), jnp.float32)),
    mesh=_mesh,
    scratch_types=[
        pltpu.VMEM((K,), jnp.int32),
        pltpu.VMEM((K, D), jnp.float32),
    ],
)
def _gather_kernel(x_hbm, r_hbm, ix_hbm, ir_hbm, ox_hbm, or_hbm, ibuf, gbuf):
    c = lax.axis_index("c")
    s = lax.axis_index("s")
    w = s * NC + c

    @pl.loop(w, GCHUNKS, step=NC * NS)
    def _(j):
        pltpu.sync_copy(ix_hbm.at[pl.ds(j * K, K)], ibuf)
        pltpu.sync_copy(x_hbm.at[ibuf], gbuf)
        pltpu.sync_copy(gbuf, ox_hbm.at[pl.ds(j * K, K)])

    @pl.loop(w, GCHUNKS, step=NC * NS)
    def _(j):
        pltpu.sync_copy(ir_hbm.at[pl.ds(j * K, K)], ibuf)
        pltpu.sync_copy(r_hbm.at[ibuf], gbuf)
        pltpu.sync_copy(gbuf, or_hbm.at[pl.ds(j * K, K)])


def _tc_prep_body(r_ref, wr1_ref, wr2_ref,
                  rneg1_ref, rneg2_ref, rfin_ref):
    r = r_ref[...]
    rneg1_ref[...] = (-ALPHA) * r
    r2 = jnp.dot(r, wr1_ref[...], preferred_element_type=jnp.float32)
    rneg2_ref[...] = (-ALPHA) * r2
    rfin_ref[...] = jnp.dot(r2, wr2_ref[...], preferred_element_type=jnp.float32)


def _tc_prep(r, wr1, wr2):
    sds = jax.ShapeDtypeStruct((N_REL, D), jnp.float32)
    return pl.pallas_call(
        _tc_prep_body,
        out_shape=(sds,) * 3,
    )(r, wr1, wr2)


def _tc_dense_body(a_ref, c_ref, x_ref,
                   win_ref, wout_ref, wloop_ref, lr_ref, b_ref, out_ref):
    norm_in = 1.0 / jnp.maximum(c_ref[0, :N_ENT, 0:1], 1.0)
    norm_out = 1.0 / jnp.maximum(c_ref[1, :N_ENT, 0:1], 1.0)
    x = x_ref[...]
    t = jnp.dot(a_ref[0, :N_ENT, :] * norm_in, win_ref[...],
                preferred_element_type=jnp.float32)
    t += jnp.dot(a_ref[1, :N_ENT, :] * norm_out, wout_ref[...],
                 preferred_element_type=jnp.float32)
    t += jnp.dot(x - lr_ref[...], wloop_ref[...],
                 preferred_element_type=jnp.float32)
    out_ref[...] = jnp.tanh(t * (1.0 / 3.0) + b_ref[...])


def _tc_dense(a, cnt, x, w_in, w_out, w_loop, loop_rel, b):
    return pl.pallas_call(
        _tc_dense_body,
        out_shape=jax.ShapeDtypeStruct((N_ENT, D), jnp.float32),
    )(a, cnt, x, w_in, w_out, w_loop, loop_rel, b.reshape(1, D))


def kernel(prop_type, ent_ix, rel_ix, quals_ix, ent_embs, rel_embs,
           edge_index, edge_type, quals,
           w_in1, w_out1, w_loop1, w_rel1, w_in2, w_out2, w_loop2, w_rel2,
           loop_rel1, loop_rel2, b1, b2):
    i32 = jnp.int32
    src = edge_index[0].astype(i32)
    dst = edge_index[1].astype(i32)
    et = edge_type.astype(i32)
    pe = HALFP - HALF
    zpe = jnp.zeros((pe,), i32)
    dpe = jnp.full((pe,), DUMMY, i32)
    zpf = jnp.zeros((3, NS * K + K), i32)
    eidx = jnp.concatenate([jnp.stack([src, et, dst]), zpf], axis=1)
    dst2 = jnp.concatenate(
        [dst[:HALF], dpe, dst[HALF:], dpe]).reshape(-1, K)

    eid = quals[2].astype(i32)
    qd = dst[eid]
    pad = NQP - NQ
    qd_in = jnp.concatenate(
        [jnp.where(eid < HALF, qd, DUMMY), jnp.full((pad,), DUMMY, i32)])
    qd_out = jnp.concatenate(
        [jnp.where(eid >= HALF, qd, DUMMY), jnp.full((pad,), DUMMY, i32)])
    qrel = jnp.concatenate([quals[0].astype(i32), jnp.zeros((pad,), i32)])
    qent = jnp.concatenate([quals[1].astype(i32), jnp.zeros((pad,), i32)])
    qidx = jnp.concatenate(
        [jnp.stack([jnp.stack([qrel, qent, qd_in]),
                    jnp.stack([qrel, qent, qd_out])]),
         jnp.zeros((2, 3, NS * K + K), i32)], axis=2)

    z128 = jnp.zeros((ZROWS, D), jnp.float32)
    ones128 = jnp.ones((K, D), jnp.float32)

    rneg1, rneg2, rfin = _tc_prep(rel_embs, w_rel1, w_rel2)

    c1 = _cnt_kernel(dst2, z128, ones128)

    a1 = _edge_kernel(ent_embs, rneg1, eidx, qidx, z128)
    x2 = _tc_dense(a1, c1, ent_embs, w_in1, w_out1, w_loop1, loop_rel1, b1)

    a2 = _edge_kernel(x2, rneg2, eidx, qidx, z128)
    x3 = _tc_dense(a2, c1, x2, w_in2, w_out2, w_loop2, loop_rel2, b2)

    idx_x = jnp.concatenate(
        [ent_ix.astype(i32), quals_ix[:, 1::2].reshape(-1).astype(i32)])
    idx_r = jnp.concatenate(
        [rel_ix.astype(i32), quals_ix[:, 0::2].reshape(-1).astype(i32)])
    gx, gr = _gather_kernel(x3, rfin, idx_x, idx_r)

    B = ent_ix.shape[0]
    sub_emb = gx[:B]
    qual_obj_emb = gx[B:].reshape(B, -1, D)
    rel_emb = gr[:B]
    qual_rel_emb = gr[B:].reshape(B, -1, D)
    return (sub_emb, rel_emb, qual_obj_emb, qual_rel_emb, x3, rfin)


# speedup vs baseline: 1.1114x; 1.1114x over previous
import functools

import jax
import jax.numpy as jnp
from jax import lax
from jax.experimental import pallas as pl
from jax.experimental.pallas import tpu as pltpu
from jax.experimental.pallas import tpu_sc as plsc

N_ENT = 10000
N_EDGE = 160000
N_REL = 400
D = 128
NQ = 40000
ALPHA = 0.8
HALF = N_EDGE // 2

NC = 2
NS = 16
K = 128
NROWS = 10112
ROWS_PER_TILE = NROWS // NS
ZROWS = 158
DUMMY = N_ENT

HALFP = 81920
E_PER_TILE = HALFP // NS
ECH_TILE = E_PER_TILE // K
ECHUNKS = HALF // K
NQP = 40960
QCHUNKS = NQP // K

_mesh = plsc.VectorSubcoreMesh(core_axis_name="c", subcore_axis_name="s")


@functools.partial(
    pl.kernel,
    out_type=jax.ShapeDtypeStruct((NC, NROWS, D), jnp.float32),
    mesh=_mesh,
    scratch_types=[
        pltpu.VMEM_SHARED((NROWS, D), jnp.float32),
        pltpu.VMEM((8, K), jnp.int32),
        pltpu.VMEM((K, D), jnp.float32),
        pltpu.SemaphoreType.DMA,
    ],
)
def _cnt_kernel(dst2_hbm, z128_hbm, ones_hbm, c_hbm, cnt, didx, ones, sem):
    c = lax.axis_index("c")
    s = lax.axis_index("s")
    r0 = s * ROWS_PER_TILE
    for j in range(ROWS_PER_TILE // ZROWS):
        pltpu.sync_copy(z128_hbm, cnt.at[pl.ds(r0 + j * ZROWS, ZROWS)])
    pltpu.sync_copy(ones_hbm, ones)
    row0 = (c * NS + s) * ECH_TILE
    plsc.subcore_barrier()

    @pl.loop(0, ECH_TILE // 8)
    def _(g):
        pltpu.sync_copy(dst2_hbm.at[pl.ds(row0 + g * 8, 8)], didx)
        ds_ = [pltpu.async_copy(ones, cnt.at[didx.at[u]], sem, add=True)
               for u in range(8)]
        for d in ds_:
            d.wait()

    plsc.subcore_barrier()
    pltpu.sync_copy(cnt.at[pl.ds(r0, ROWS_PER_TILE)],
                    c_hbm.at[c, pl.ds(r0, ROWS_PER_TILE)])


@functools.partial(
    pl.kernel,
    out_type=jax.ShapeDtypeStruct((NC, NROWS, D), jnp.float32),
    mesh=_mesh,
    scratch_types=[
        pltpu.VMEM_SHARED((NROWS, D), jnp.float32),
        pltpu.VMEM_SHARED((N_REL, D), jnp.float32),
        pltpu.VMEM((K, D), jnp.float32),
        pltpu.VMEM((K, D), jnp.float32),
        pltpu.VMEM((3, K), jnp.int32),
        pltpu.SemaphoreType.DMA,
        pltpu.SemaphoreType.DMA,
    ],
)
def _edge_kernel(x_hbm, rneg_hbm, eidx_hbm, qidx_hbm, z128_hbm,
                 a_hbm, acc, rtab, xbuf, rbuf, ibuf, semx, semr):
    c = lax.axis_index("c")
    s = lax.axis_index("s")

    r0 = s * ROWS_PER_TILE
    for j in range(ROWS_PER_TILE // ZROWS):
        pltpu.sync_copy(z128_hbm, acc.at[pl.ds(r0 + j * ZROWS, ZROWS)])

    @pl.when(s == 0)
    def _():
        pltpu.sync_copy(rneg_hbm, rtab)

    plsc.subcore_barrier()

    @pl.loop(s, ECHUNKS, step=NS)
    def _(j):
        pltpu.sync_copy(eidx_hbm.at[:, pl.ds(c * HALF + j * K, K)], ibuf)
        gx = pltpu.async_copy(x_hbm.at[ibuf.at[0]], xbuf, semx)
        gr = pltpu.async_copy(rtab.at[ibuf.at[1]], rbuf, semr)
        gx.wait()
        sx = pltpu.async_copy(xbuf, acc.at[ibuf.at[2]], semx, add=True)
        gr.wait()
        sr = pltpu.async_copy(rbuf, acc.at[ibuf.at[2]], semr, add=True)
        sx.wait()
        sr.wait()

    @pl.loop(s, QCHUNKS, step=NS)
    def _(j):
        pltpu.sync_copy(qidx_hbm.at[c, :, pl.ds(j * K, K)], ibuf)
        gr = pltpu.async_copy(rtab.at[ibuf.at[0]], rbuf, semr)
        gx = pltpu.async_copy(x_hbm.at[ibuf.at[1]], xbuf, semx)
        gr.wait()
        gx.wait()

        @plsc.parallel_loop(0, K * D, step=16, unroll=8)
        def _(t):
            i = t // D
            k = t - i * D
            xbuf[i, pl.ds(k, 16)] = (
                xbuf[i, pl.ds(k, 16)] * rbuf[i, pl.ds(k, 16)]) * 0.25

        pltpu.sync_copy(xbuf, acc.at[ibuf.at[2]], add=True)

    plsc.subcore_barrier()
    pltpu.sync_copy(acc.at[pl.ds(r0, ROWS_PER_TILE)],
                    a_hbm.at[c, pl.ds(r0, ROWS_PER_TILE)])


NGX = 7168
GCHUNKS = NGX // K


@functools.partial(
    pl.kernel,
    out_type=(jax.ShapeDtypeStruct((NGX, D), jnp.float32),
              jax.ShapeDtypeStruct((NGX, D), jnp.float32)),
    mesh=_mesh,
    scratch_types=[
        pltpu.VMEM((K,), jnp.int32),
        pltpu.VMEM((K, D), jnp.float32),
    ],
)
def _gather_kernel(x_hbm, r_hbm, ix_hbm, ir_hbm, ox_hbm, or_hbm, ibuf, gbuf):
    c = lax.axis_index("c")
    s = lax.axis_index("s")
    w = s * NC + c

    @pl.loop(w, GCHUNKS, step=NC * NS)
    def _(j):
        pltpu.sync_copy(ix_hbm.at[pl.ds(j * K, K)], ibuf)
        pltpu.sync_copy(x_hbm.at[ibuf], gbuf)
        pltpu.sync_copy(gbuf, ox_hbm.at[pl.ds(j * K, K)])

    @pl.loop(w, GCHUNKS, step=NC * NS)
    def _(j):
        pltpu.sync_copy(ir_hbm.at[pl.ds(j * K, K)], ibuf)
        pltpu.sync_copy(r_hbm.at[ibuf], gbuf)
        pltpu.sync_copy(gbuf, or_hbm.at[pl.ds(j * K, K)])


def _tc_prep_body(r_ref, wr1_ref, wr2_ref,
                  rneg1_ref, rneg2_ref, rfin_ref):
    r = r_ref[...]
    rneg1_ref[...] = (-ALPHA) * r
    r2 = jnp.dot(r, wr1_ref[...], preferred_element_type=jnp.float32)
    rneg2_ref[...] = (-ALPHA) * r2
    rfin_ref[...] = jnp.dot(r2, wr2_ref[...], preferred_element_type=jnp.float32)


def _tc_prep(r, wr1, wr2):
    sds = jax.ShapeDtypeStruct((N_REL, D), jnp.float32)
    return pl.pallas_call(
        _tc_prep_body,
        out_shape=(sds,) * 3,
    )(r, wr1, wr2)


def _tc_dense_body(a_ref, c_ref, x_ref,
                   win_ref, wout_ref, wloop_ref, lr_ref, b_ref, out_ref):
    norm_in = 1.0 / jnp.maximum(c_ref[0, :N_ENT, 0:1], 1.0)
    norm_out = 1.0 / jnp.maximum(c_ref[1, :N_ENT, 0:1], 1.0)
    x = x_ref[...]
    t = jnp.dot(a_ref[0, :N_ENT, :] * norm_in, win_ref[...],
                preferred_element_type=jnp.float32)
    t += jnp.dot(a_ref[1, :N_ENT, :] * norm_out, wout_ref[...],
                 preferred_element_type=jnp.float32)
    t += jnp.dot(x - lr_ref[...], wloop_ref[...],
                 preferred_element_type=jnp.float32)
    out_ref[...] = jnp.tanh(t * (1.0 / 3.0) + b_ref[...])


def _tc_dense(a, cnt, x, w_in, w_out, w_loop, loop_rel, b):
    return pl.pallas_call(
        _tc_dense_body,
        out_shape=jax.ShapeDtypeStruct((N_ENT, D), jnp.float32),
    )(a, cnt, x, w_in, w_out, w_loop, loop_rel, b.reshape(1, D))


def kernel(prop_type, ent_ix, rel_ix, quals_ix, ent_embs, rel_embs,
           edge_index, edge_type, quals,
           w_in1, w_out1, w_loop1, w_rel1, w_in2, w_out2, w_loop2, w_rel2,
           loop_rel1, loop_rel2, b1, b2):
    i32 = jnp.int32
    src = edge_index[0].astype(i32)
    dst = edge_index[1].astype(i32)
    et = edge_type.astype(i32)
    pe = HALFP - HALF
    zpe = jnp.zeros((pe,), i32)
    dpe = jnp.full((pe,), DUMMY, i32)
    eidx = jnp.stack([src, et, dst])
    dst2 = jnp.concatenate(
        [dst[:HALF], dpe, dst[HALF:], dpe]).reshape(-1, K)

    eid = quals[2].astype(i32)
    qd = dst[eid]
    pad = NQP - NQ
    qd_in = jnp.concatenate(
        [jnp.where(eid < HALF, qd, DUMMY), jnp.full((pad,), DUMMY, i32)])
    qd_out = jnp.concatenate(
        [jnp.where(eid >= HALF, qd, DUMMY), jnp.full((pad,), DUMMY, i32)])
    qrel = jnp.concatenate([quals[0].astype(i32), jnp.zeros((pad,), i32)])
    qent = jnp.concatenate([quals[1].astype(i32), jnp.zeros((pad,), i32)])
    qidx = jnp.stack([jnp.stack([qrel, qent, qd_in]),
                      jnp.stack([qrel, qent, qd_out])])

    z128 = jnp.zeros((ZROWS, D), jnp.float32)
    ones128 = jnp.ones((K, D), jnp.float32)

    rneg1, rneg2, rfin = _tc_prep(rel_embs, w_rel1, w_rel2)

    c1 = _cnt_kernel(dst2, z128, ones128)

    a1 = _edge_kernel(ent_embs, rneg1, eidx, qidx, z128)
    x2 = _tc_dense(a1, c1, ent_embs, w_in1, w_out1, w_loop1, loop_rel1, b1)

    a2 = _edge_kernel(x2, rneg2, eidx, qidx, z128)
    x3 = _tc_dense(a2, c1, x2, w_in2, w_out2, w_loop2, loop_rel2, b2)

    idx_x = jnp.concatenate(
        [ent_ix.astype(i32), quals_ix[:, 1::2].reshape(-1).astype(i32)])
    idx_r = jnp.concatenate(
        [rel_ix.astype(i32), quals_ix[:, 0::2].reshape(-1).astype(i32)])
    gx, gr = _gather_kernel(x3, rfin, idx_x, idx_r)

    B = ent_ix.shape[0]
    sub_emb = gx[:B]
    qual_obj_emb = gx[B:].reshape(B, -1, D)
    rel_emb = gr[:B]
    qual_rel_emb = gr[B:].reshape(B, -1, D)
    return (sub_emb, rel_emb, qual_obj_emb, qual_rel_emb, x3, rfin)
